# baseline (speedup 1.0000x reference)
import functools
import jax
import jax.numpy as jnp
from jax import lax
from jax.experimental import pallas as pl
from jax.experimental.pallas import tpu as pltpu
from jax.experimental.pallas import tpu_sc as plsc

_R = 128
_N = 32768
_NW = 32
_ROWS_PER_W = _R // _NW


def kernel(x):
    mesh = plsc.VectorSubcoreMesh(core_axis_name="c", subcore_axis_name="s")

    @functools.partial(
        pl.kernel,
        mesh=mesh,
        out_type=jax.ShapeDtypeStruct((_R, _N), jnp.float32),
        scratch_types=[pltpu.VMEM((_N,), jnp.float32)],
    )
    def _sc_scan(x_hbm, o_hbm, row_v):
        wid = lax.axis_index("s") * 2 + lax.axis_index("c")
        for r in range(_ROWS_PER_W):
            row = wid * _ROWS_PER_W + r
            pltpu.sync_copy(x_hbm.at[row], row_v)

            lane = lax.iota(jnp.int32, 16)
            idx15 = lane * 0 + 15
            dnums = lax.GatherDimensionNumbers(
                offset_dims=(), collapsed_slice_dims=(0,),
                start_index_map=(0,))

            def gat(v, idx):
                return lax.gather(
                    v, idx[:, None], dimension_numbers=dnums,
                    slice_sizes=(1,),
                    mode=lax.GatherScatterMode.PROMISE_IN_BOUNDS)

            zero = jnp.zeros((16,), jnp.float32)

            def body(k, carry):
                v = row_v[pl.ds(k * 16, 16)]
                for p in (1, 2, 4, 8):
                    shifted = gat(v, jnp.maximum(lane - p, 0))
                    v = v + jnp.where(lane >= p, shifted, zero)
                s = v + carry
                row_v[pl.ds(k * 16, 16)] = s
                return gat(s, idx15)

            lax.fori_loop(0, _N // 16, body, zero)
            pltpu.sync_copy(row_v, o_hbm.at[row])

    return _sc_scan(x)
